# Initial kernel scaffold; baseline (speedup 1.0000x reference)
import jax
import jax.numpy as jnp
from jax.experimental import pallas as pl


def kernel(x, gc1_att, gc1_weight_seq, gc1_weight_c, gc1_bias, bn1_gamma, bn1_beta, bn1_mean, bn1_var, gc7_att, gc7_weight_seq, gc7_weight_c, gc7_bias, conv_weight, conv_bias, gcb0_gc1_att, gcb0_gc1_weight_seq, gcb0_gc1_weight_c, gcb0_gc1_bias, gcb0_bn1_gamma, gcb0_bn1_beta, gcb0_bn1_mean, gcb0_bn1_var, gcb0_gc2_att, gcb0_gc2_weight_seq, gcb0_gc2_weight_c, gcb0_gc2_bias, gcb0_bn2_gamma, gcb0_bn2_beta, gcb0_bn2_mean, gcb0_bn2_var, gcb1_gc1_att, gcb1_gc1_weight_seq, gcb1_gc1_weight_c, gcb1_gc1_bias, gcb1_bn1_gamma, gcb1_bn1_beta, gcb1_bn1_mean, gcb1_bn1_var, gcb1_gc2_att, gcb1_gc2_weight_seq, gcb1_gc2_weight_c, gcb1_gc2_bias, gcb1_bn2_gamma, gcb1_bn2_beta, gcb1_bn2_mean, gcb1_bn2_var):
    raise NotImplementedError("write your pallas kernel here")


# speedup vs baseline: 3.2268x; 3.2268x over previous
import jax
import jax.numpy as jnp
from jax.experimental import pallas as pl
from jax.experimental.pallas import tpu as pltpu

_GROUP = 4


def _decoder_body(x_ref, attbd_ref, w2_ref, bns_ref, bnb_ref,
                  att7_ref, w27_ref, wconv_ref, b7_ref, o_ref):
    num_hidden = attbd_ref.shape[0]
    num_stage = (num_hidden - 1) // 2
    gn = attbd_ref.shape[1]
    n_groups = x_ref.shape[0] // gn
    bf16 = jnp.bfloat16

    def gc_bn_tanh(acts, k):
        out = []
        for g in range(n_groups):
            t = jnp.dot(attbd_ref[k], acts[g],
                        preferred_element_type=jnp.float32)
            u = jnp.dot(t.astype(bf16), w2_ref[k],
                        preferred_element_type=jnp.float32)
            out.append(jnp.tanh(u * bns_ref[k] + bnb_ref[k]))
        return out

    xg = [x_ref[g * gn:(g + 1) * gn, :] for g in range(n_groups)]
    y = gc_bn_tanh(xg, 0)
    for s in range(num_stage):
        a = gc_bn_tanh([v.astype(bf16) for v in y], 1 + 2 * s)
        b = gc_bn_tanh([v.astype(bf16) for v in a], 2 + 2 * s)
        y = [bv + yv for bv, yv in zip(b, y)]

    for g in range(n_groups):
        t = jnp.dot(att7_ref[...], y[g].astype(bf16),
                    preferred_element_type=jnp.float32)
        u = jnp.dot(t.astype(bf16), w27_ref[...],
                    preferred_element_type=jnp.float32)
        u = u + jnp.dot(xg[g], wconv_ref[...],
                        preferred_element_type=jnp.float32)
        o_ref[g * gn:(g + 1) * gn, :] = u + b7_ref[...]


def _kron_weight(wc, ws):
    C, OC = wc.shape
    L = ws.shape[0]
    return jnp.einsum("co,lm->clom", wc, ws).reshape(C * L, OC * L)


def _bn_fold(gamma, beta, mean, var, bias_row, C, N, L, eps=1e-5):
    inv_std = 1.0 / jnp.sqrt(var + eps)
    scale = (gamma * inv_std).reshape(C, N, L)
    shift = (beta - mean * gamma * inv_std).reshape(C, N, L)
    scale2d = jnp.transpose(scale, (1, 0, 2)).reshape(N, C * L)
    shift2d = jnp.transpose(shift, (1, 0, 2)).reshape(N, C * L)
    return scale2d, bias_row * scale2d + shift2d


def kernel(
    x,
    gc1_att, gc1_weight_seq, gc1_weight_c, gc1_bias,
    bn1_gamma, bn1_beta, bn1_mean, bn1_var,
    gc7_att, gc7_weight_seq, gc7_weight_c, gc7_bias,
    conv_weight, conv_bias,
    gcb0_gc1_att, gcb0_gc1_weight_seq, gcb0_gc1_weight_c, gcb0_gc1_bias,
    gcb0_bn1_gamma, gcb0_bn1_beta, gcb0_bn1_mean, gcb0_bn1_var,
    gcb0_gc2_att, gcb0_gc2_weight_seq, gcb0_gc2_weight_c, gcb0_gc2_bias,
    gcb0_bn2_gamma, gcb0_bn2_beta, gcb0_bn2_mean, gcb0_bn2_var,
    gcb1_gc1_att, gcb1_gc1_weight_seq, gcb1_gc1_weight_c, gcb1_gc1_bias,
    gcb1_bn1_gamma, gcb1_bn1_beta, gcb1_bn1_mean, gcb1_bn1_var,
    gcb1_gc2_att, gcb1_gc2_weight_seq, gcb1_gc2_weight_c, gcb1_gc2_bias,
    gcb1_bn2_gamma, gcb1_bn2_beta, gcb1_bn2_mean, gcb1_bn2_var,
):
    B, C, N, L = x.shape
    CL = C * L
    OC = gc7_weight_c.shape[1]
    OCL = OC * L
    bf16 = jnp.bfloat16

    hidden = [
        (gc1_att, gc1_weight_seq, gc1_weight_c, gc1_bias,
         bn1_gamma, bn1_beta, bn1_mean, bn1_var),
        (gcb0_gc1_att, gcb0_gc1_weight_seq, gcb0_gc1_weight_c, gcb0_gc1_bias,
         gcb0_bn1_gamma, gcb0_bn1_beta, gcb0_bn1_mean, gcb0_bn1_var),
        (gcb0_gc2_att, gcb0_gc2_weight_seq, gcb0_gc2_weight_c, gcb0_gc2_bias,
         gcb0_bn2_gamma, gcb0_bn2_beta, gcb0_bn2_mean, gcb0_bn2_var),
        (gcb1_gc1_att, gcb1_gc1_weight_seq, gcb1_gc1_weight_c, gcb1_gc1_bias,
         gcb1_bn1_gamma, gcb1_bn1_beta, gcb1_bn1_mean, gcb1_bn1_var),
        (gcb1_gc2_att, gcb1_gc2_weight_seq, gcb1_gc2_weight_c, gcb1_gc2_bias,
         gcb1_bn2_gamma, gcb1_bn2_beta, gcb1_bn2_mean, gcb1_bn2_var),
    ]
    NH = len(hidden)

    eye_g = jnp.eye(_GROUP, dtype=jnp.float32)
    GN = _GROUP * N

    attbd_h, w2_h, bns_h, bnb_h = [], [], [], []
    for (att, ws, wc, bias, g_, b_, m_, v_) in hidden:
        attbd_h.append(jnp.kron(eye_g, att).astype(bf16))
        w2_h.append(_kron_weight(wc, ws).astype(bf16))
        bias_row = jnp.tile(bias, (C,)).reshape(1, CL)
        s2d, sh2d = _bn_fold(g_, b_, m_, v_, bias_row, C, N, L)
        bns_h.append(jnp.tile(s2d, (_GROUP, 1)))
        bnb_h.append(jnp.tile(sh2d, (_GROUP, 1)))
    attbd_h = jnp.stack(attbd_h)
    w2_h = jnp.stack(w2_h)
    bns_h = jnp.stack(bns_h)
    bnb_h = jnp.stack(bnb_h)

    att7bd = jnp.kron(eye_g, gc7_att).astype(bf16)
    w27 = _kron_weight(gc7_weight_c, gc7_weight_seq).astype(bf16)
    eye_l = jnp.eye(L, dtype=jnp.float32)
    wconv = jnp.einsum("oc,lm->clom", conv_weight, eye_l).reshape(CL, OCL)
    wconv = wconv.astype(bf16)
    b7 = (jnp.tile(gc7_bias, (OC,)) + jnp.repeat(conv_bias, L)).reshape(1, OCL)

    x2d = jnp.transpose(x, (0, 2, 1, 3)).reshape(B * N, CL).astype(bf16)

    BB = 16 if B % 16 == 0 else _GROUP
    ROWS = BB * N
    grid = (B // BB,)

    out2d = pl.pallas_call(
        _decoder_body,
        out_shape=jax.ShapeDtypeStruct((B * N, OCL), jnp.float32),
        grid=grid,
        in_specs=[
            pl.BlockSpec((ROWS, CL), lambda i: (i, 0)),
            pl.BlockSpec((NH, GN, GN), lambda i: (0, 0, 0)),
            pl.BlockSpec((NH, CL, CL), lambda i: (0, 0, 0)),
            pl.BlockSpec((NH, GN, CL), lambda i: (0, 0, 0)),
            pl.BlockSpec((NH, GN, CL), lambda i: (0, 0, 0)),
            pl.BlockSpec((GN, GN), lambda i: (0, 0)),
            pl.BlockSpec((CL, OCL), lambda i: (0, 0)),
            pl.BlockSpec((CL, OCL), lambda i: (0, 0)),
            pl.BlockSpec((1, OCL), lambda i: (0, 0)),
        ],
        out_specs=pl.BlockSpec((ROWS, OCL), lambda i: (i, 0)),
        compiler_params=pltpu.CompilerParams(
            dimension_semantics=("parallel",)),
    )(x2d, attbd_h, w2_h, bns_h, bnb_h, att7bd, w27, wconv, b7)

    return jnp.transpose(out2d.reshape(B, N, OC, L), (0, 2, 1, 3))


# speedup vs baseline: 3.8106x; 1.1809x over previous
import jax
import jax.numpy as jnp
from jax.experimental import pallas as pl
from jax.experimental.pallas import tpu as pltpu

_GROUP = 4


def _decoder_body(x_ref, attbd_ref, w2_ref, bns_ref, bnb_ref,
                  att7_ref, w27_ref, wconv_ref, b7_ref, o_ref):
    num_hidden = attbd_ref.shape[0]
    num_stage = (num_hidden - 1) // 2
    gn = attbd_ref.shape[1]
    n, l = x_ref.shape[1], x_ref.shape[2]
    g_batch = gn // n
    bf16 = jnp.bfloat16

    n_c = w2_ref.shape[1] // l
    oc = w27_ref.shape[1] // l
    bb = x_ref.shape[0] // n_c
    n_groups = bb // g_batch

    xg = []
    for g in range(n_groups):
        rows = []
        for b in range(g_batch):
            bi = g * g_batch + b
            rows.append(jnp.concatenate(
                [x_ref[bi * n_c + c] for c in range(n_c)], axis=1))
        xg.append(jnp.concatenate(rows, axis=0).astype(bf16))

    def gc_bn_tanh(acts, k):
        out = []
        for g in range(n_groups):
            t = jnp.dot(attbd_ref[k], acts[g],
                        preferred_element_type=jnp.float32)
            u = jnp.dot(t.astype(bf16), w2_ref[k],
                        preferred_element_type=jnp.float32)
            out.append(jnp.tanh(u * bns_ref[k] + bnb_ref[k]))
        return out

    y = gc_bn_tanh(xg, 0)
    for s in range(num_stage):
        a = gc_bn_tanh([v.astype(bf16) for v in y], 1 + 2 * s)
        b = gc_bn_tanh([v.astype(bf16) for v in a], 2 + 2 * s)
        y = [bv + yv for bv, yv in zip(b, y)]

    for g in range(n_groups):
        t = jnp.dot(att7_ref[...], y[g].astype(bf16),
                    preferred_element_type=jnp.float32)
        u = jnp.dot(t.astype(bf16), w27_ref[...],
                    preferred_element_type=jnp.float32)
        u = u + jnp.dot(xg[g], wconv_ref[...],
                        preferred_element_type=jnp.float32)
        u = u + b7_ref[...]
        for b in range(g_batch):
            bi = g * g_batch + b
            for c in range(oc):
                o_ref[bi * oc + c] = u[b * n:(b + 1) * n,
                                       c * l:(c + 1) * l]


def _kron_weight(wc, ws):
    C, OC = wc.shape
    L = ws.shape[0]
    return jnp.einsum("co,lm->clom", wc, ws).reshape(C * L, OC * L)


def _bn_fold(gamma, beta, mean, var, bias_row, C, N, L, eps=1e-5):
    inv_std = 1.0 / jnp.sqrt(var + eps)
    scale = (gamma * inv_std).reshape(C, N, L)
    shift = (beta - mean * gamma * inv_std).reshape(C, N, L)
    scale2d = jnp.transpose(scale, (1, 0, 2)).reshape(N, C * L)
    shift2d = jnp.transpose(shift, (1, 0, 2)).reshape(N, C * L)
    return scale2d, bias_row * scale2d + shift2d


def kernel(
    x,
    gc1_att, gc1_weight_seq, gc1_weight_c, gc1_bias,
    bn1_gamma, bn1_beta, bn1_mean, bn1_var,
    gc7_att, gc7_weight_seq, gc7_weight_c, gc7_bias,
    conv_weight, conv_bias,
    gcb0_gc1_att, gcb0_gc1_weight_seq, gcb0_gc1_weight_c, gcb0_gc1_bias,
    gcb0_bn1_gamma, gcb0_bn1_beta, gcb0_bn1_mean, gcb0_bn1_var,
    gcb0_gc2_att, gcb0_gc2_weight_seq, gcb0_gc2_weight_c, gcb0_gc2_bias,
    gcb0_bn2_gamma, gcb0_bn2_beta, gcb0_bn2_mean, gcb0_bn2_var,
    gcb1_gc1_att, gcb1_gc1_weight_seq, gcb1_gc1_weight_c, gcb1_gc1_bias,
    gcb1_bn1_gamma, gcb1_bn1_beta, gcb1_bn1_mean, gcb1_bn1_var,
    gcb1_gc2_att, gcb1_gc2_weight_seq, gcb1_gc2_weight_c, gcb1_gc2_bias,
    gcb1_bn2_gamma, gcb1_bn2_beta, gcb1_bn2_mean, gcb1_bn2_var,
):
    B, C, N, L = x.shape
    CL = C * L
    OC = gc7_weight_c.shape[1]
    OCL = OC * L
    bf16 = jnp.bfloat16

    hidden = [
        (gc1_att, gc1_weight_seq, gc1_weight_c, gc1_bias,
         bn1_gamma, bn1_beta, bn1_mean, bn1_var),
        (gcb0_gc1_att, gcb0_gc1_weight_seq, gcb0_gc1_weight_c, gcb0_gc1_bias,
         gcb0_bn1_gamma, gcb0_bn1_beta, gcb0_bn1_mean, gcb0_bn1_var),
        (gcb0_gc2_att, gcb0_gc2_weight_seq, gcb0_gc2_weight_c, gcb0_gc2_bias,
         gcb0_bn2_gamma, gcb0_bn2_beta, gcb0_bn2_mean, gcb0_bn2_var),
        (gcb1_gc1_att, gcb1_gc1_weight_seq, gcb1_gc1_weight_c, gcb1_gc1_bias,
         gcb1_bn1_gamma, gcb1_bn1_beta, gcb1_bn1_mean, gcb1_bn1_var),
        (gcb1_gc2_att, gcb1_gc2_weight_seq, gcb1_gc2_weight_c, gcb1_gc2_bias,
         gcb1_bn2_gamma, gcb1_bn2_beta, gcb1_bn2_mean, gcb1_bn2_var),
    ]
    NH = len(hidden)

    eye_g = jnp.eye(_GROUP, dtype=jnp.float32)
    GN = _GROUP * N

    attbd_h, w2_h, bns_h, bnb_h = [], [], [], []
    for (att, ws, wc, bias, g_, b_, m_, v_) in hidden:
        attbd_h.append(jnp.kron(eye_g, att).astype(bf16))
        w2_h.append(_kron_weight(wc, ws).astype(bf16))
        bias_row = jnp.tile(bias, (C,)).reshape(1, CL)
        s2d, sh2d = _bn_fold(g_, b_, m_, v_, bias_row, C, N, L)
        bns_h.append(jnp.tile(s2d, (_GROUP, 1)))
        bnb_h.append(jnp.tile(sh2d, (_GROUP, 1)))
    attbd_h = jnp.stack(attbd_h)
    w2_h = jnp.stack(w2_h)
    bns_h = jnp.stack(bns_h)
    bnb_h = jnp.stack(bnb_h)

    att7bd = jnp.kron(eye_g, gc7_att).astype(bf16)
    w27 = _kron_weight(gc7_weight_c, gc7_weight_seq).astype(bf16)
    eye_l = jnp.eye(L, dtype=jnp.float32)
    wconv = jnp.einsum("oc,lm->clom", conv_weight, eye_l).reshape(CL, OCL)
    wconv = wconv.astype(bf16)
    b7 = (jnp.tile(gc7_bias, (OC,)) + jnp.repeat(conv_bias, L)).reshape(1, OCL)

    x3 = x.reshape(B * C, N, L)

    BB = 16 if B % 16 == 0 else _GROUP
    grid = (B // BB,)

    out3 = pl.pallas_call(
        _decoder_body,
        out_shape=jax.ShapeDtypeStruct((B * OC, N, L), jnp.float32),
        grid=grid,
        in_specs=[
            pl.BlockSpec((BB * C, N, L), lambda i: (i, 0, 0)),
            pl.BlockSpec((NH, GN, GN), lambda i: (0, 0, 0)),
            pl.BlockSpec((NH, CL, CL), lambda i: (0, 0, 0)),
            pl.BlockSpec((NH, GN, CL), lambda i: (0, 0, 0)),
            pl.BlockSpec((NH, GN, CL), lambda i: (0, 0, 0)),
            pl.BlockSpec((GN, GN), lambda i: (0, 0)),
            pl.BlockSpec((CL, OCL), lambda i: (0, 0)),
            pl.BlockSpec((CL, OCL), lambda i: (0, 0)),
            pl.BlockSpec((1, OCL), lambda i: (0, 0)),
        ],
        out_specs=pl.BlockSpec((BB * OC, N, L), lambda i: (i, 0, 0)),
        compiler_params=pltpu.CompilerParams(
            dimension_semantics=("parallel",)),
    )(x3, attbd_h, w2_h, bns_h, bnb_h, att7bd, w27, wconv, b7)

    return out3.reshape(B, OC, N, L)
